# baseline (speedup 1.0000x reference)
import functools

import jax
import jax.numpy as jnp
from jax import lax
from jax.experimental import pallas as pl
from jax.experimental.pallas import tpu as pltpu
from jax.experimental.pallas import tpu_sc as plsc

B, N, M = 8, 4096, 1024
C1, C2 = 256, 512
CO = 256
HB = B // 2

TN = 1024
TN2 = 1024
NT = N // TN
NT2 = N // TN2

NW = 32
PPW = (HB * N) // NW
CH = 16
NCH = PPW // CH
K = 3


def _knn_body(u_ref, k_ref, kf_ref, i0_ref, i1_ref, i2_ref,
              w0_ref, w1_ref, w2_ref, kft_ref):
    b = pl.program_id(0)
    u = u_ref[0]
    kp = k_ref[0]

    @pl.when(pl.program_id(1) == 0)
    def _():
        kft_ref[...] = jnp.swapaxes(kf_ref[0], 0, 1)

    uk = lax.dot_general(u, kp, (((1,), (1,)), ((), ())),
                         preferred_element_type=jnp.float32)
    u2 = jnp.sum(u * u, axis=1, keepdims=True)
    k2 = jnp.sum(kp * kp, axis=1)[None, :]
    d = jnp.maximum(u2 + k2 - 2.0 * uk, 0.0)
    iota = lax.broadcasted_iota(jnp.int32, d.shape, 1).astype(jnp.float32)
    idxs, recips = [], []
    for k in range(K):
        mk = jnp.min(d, axis=1, keepdims=True)
        am = jnp.min(jnp.where(d == mk, iota, jnp.float32(M)),
                     axis=1, keepdims=True)
        idxs.append(am)
        recips.append(1.0 / (jnp.sqrt(mk) + 1e-8))
        if k < K - 1:
            d = jnp.where(iota == am, jnp.float32(jnp.inf), d)
    norm = recips[0] + recips[1] + recips[2]
    for am, r, i_ref, wr_ref in zip(idxs, recips,
                                    (i0_ref, i1_ref, i2_ref),
                                    (w0_ref, w1_ref, w2_ref)):
        i_ref[...] = jnp.squeeze(am.astype(jnp.int32) + b * M, -1)
        wr_ref[...] = jnp.broadcast_to(r / norm, (TN, 16))


def _knn(unknown, known, known_feats, h):
    return pl.pallas_call(
        _knn_body,
        grid=(HB, NT),
        in_specs=[
            pl.BlockSpec((1, TN, 3), lambda b, t: (b + h * HB, t, 0)),
            pl.BlockSpec((1, M, 3), lambda b, t: (b + h * HB, 0, 0)),
            pl.BlockSpec((1, C2, M), lambda b, t: (b + h * HB, 0, 0)),
        ],
        out_specs=[
            pl.BlockSpec((TN,), lambda b, t: (b * NT + t,)),
            pl.BlockSpec((TN,), lambda b, t: (b * NT + t,)),
            pl.BlockSpec((TN,), lambda b, t: (b * NT + t,)),
            pl.BlockSpec((TN, 16), lambda b, t: (b * NT + t, 0)),
            pl.BlockSpec((TN, 16), lambda b, t: (b * NT + t, 0)),
            pl.BlockSpec((TN, 16), lambda b, t: (b * NT + t, 0)),
            pl.BlockSpec((M, C2), lambda b, t: (b, 0)),
        ],
        out_shape=[
            jax.ShapeDtypeStruct((HB * N,), jnp.int32),
            jax.ShapeDtypeStruct((HB * N,), jnp.int32),
            jax.ShapeDtypeStruct((HB * N,), jnp.int32),
            jax.ShapeDtypeStruct((HB * N, 16), jnp.float32),
            jax.ShapeDtypeStruct((HB * N, 16), jnp.float32),
            jax.ShapeDtypeStruct((HB * N, 16), jnp.float32),
            jax.ShapeDtypeStruct((HB * M, C2), jnp.float32),
        ],
    )(unknown, known, known_feats)


def _sc_interp_body(i0_hbm, i1_hbm, i2_hbm, w0_hbm, w1_hbm, w2_hbm,
                    kft_hbm, out_hbm,
                    idx0, idx1, wva0, wvb0, wvc0, wva1, wvb1, wvc1,
                    rows0, rows1, out0, out1,
                    gsem0, gsem1, osem0, osem1, iwsem0, iwsem1):
    wid = lax.axis_index("s") * 2 + lax.axis_index("c")
    base = wid * PPW
    i_hbms = (i0_hbm, i1_hbm, i2_hbm)
    w_hbms = (w0_hbm, w1_hbm, w2_hbm)

    def _iw_copy(c, idx_v, wvs, sem):
        pb = base + c * CH
        for k in range(K):
            pltpu.async_copy(i_hbms[k].at[pl.ds(pb, CH)],
                             idx_v.at[pl.ds(k * CH, CH)], sem)
            pltpu.async_copy(w_hbms[k].at[pl.ds(pb, CH)], wvs[k], sem)

    def _wait_iw(idx_v, wvs, sem):
        for k in range(K):
            pltpu.make_async_copy(i_hbms[k].at[pl.ds(0, CH)],
                                  idx_v.at[pl.ds(k * CH, CH)], sem).wait()
            pltpu.make_async_copy(w_hbms[k].at[pl.ds(0, CH)],
                                  wvs[k], sem).wait()

    def _gather(idx_v, rows_v, sem):
        for k in range(K):
            pltpu.async_copy(kft_hbm.at[idx_v.at[pl.ds(k * CH, CH)]],
                             rows_v.at[pl.ds(k * CH, CH)], sem)

    def _wait_gather(rows_v, sem):
        pltpu.make_async_copy(kft_hbm.at[pl.ds(0, CH * K)], rows_v, sem).wait()

    def _compute(rows_v, wvs, out_v):
        wva, wvb, wvc = wvs

        @plsc.parallel_loop(0, CH, 1)
        def point(j):
            wa = wva[j, :]
            wb = wvb[j, :]
            wc = wvc[j, :]
            for c in range(C2 // 16):
                sl = pl.ds(c * 16, 16)
                acc = wa * rows_v[j, sl]
                acc = acc + wb * rows_v[CH + j, sl]
                acc = acc + wc * rows_v[2 * CH + j, sl]
                out_v[j, sl] = acc

    def _wait_store(out_v, sem):
        pltpu.make_async_copy(out_hbm.at[pl.ds(0, CH)], out_v, sem).wait()

    wvs0 = (wva0, wvb0, wvc0)
    wvs1 = (wva1, wvb1, wvc1)

    _iw_copy(0, idx0, wvs0, iwsem0)
    _wait_iw(idx0, wvs0, iwsem0)
    _gather(idx0, rows0, gsem0)
    _iw_copy(1, idx1, wvs1, iwsem1)

    def pair(i, carry):
        c0 = 2 * i
        c1 = 2 * i + 1
        last = NCH // 2 - 1
        _wait_iw(idx1, wvs1, iwsem1)
        _gather(idx1, rows1, gsem1)
        _wait_gather(rows0, gsem0)

        @pl.when(i > 0)
        def _():
            _wait_store(out0, osem0)

        _compute(rows0, wvs0, out0)
        pltpu.async_copy(out0, out_hbm.at[pl.ds(base + c0 * CH, CH)], osem0)

        @pl.when(i < last)
        def _():
            _iw_copy(c0 + 2, idx0, wvs0, iwsem0)

        _wait_gather(rows1, gsem1)

        @pl.when(i < last)
        def _():
            _wait_iw(idx0, wvs0, iwsem0)
            _gather(idx0, rows0, gsem0)

        @pl.when(i > 0)
        def _():
            _wait_store(out1, osem1)

        _compute(rows1, wvs1, out1)
        pltpu.async_copy(out1, out_hbm.at[pl.ds(base + c1 * CH, CH)], osem1)

        @pl.when(i < last)
        def _():
            _iw_copy(c1 + 2, idx1, wvs1, iwsem1)

        return carry

    lax.fori_loop(0, NCH // 2, pair, 0)
    _wait_store(out0, osem0)
    _wait_store(out1, osem1)


def _sc_interp(i0, i1, i2, w0e, w1e, w2e, kft):
    mesh = plsc.VectorSubcoreMesh(core_axis_name="c", subcore_axis_name="s")
    f = functools.partial(
        pl.kernel,
        out_type=jax.ShapeDtypeStruct((HB * N, C2), jnp.float32),
        mesh=mesh,
        scratch_types=(
            [pltpu.VMEM((CH * K,), jnp.int32)] * 2
            + [pltpu.VMEM((CH, 16), jnp.float32)] * 6
            + [pltpu.VMEM((CH * K, C2), jnp.float32)] * 2
            + [pltpu.VMEM((CH, C2), jnp.float32)] * 2
            + [pltpu.SemaphoreType.DMA] * 6
        ),
    )(_sc_interp_body)
    return f(i0, i1, i2, w0e, w1e, w2e, kft)


def _mlp1_body(it_ref, uf_ref, w0_ref, y1_ref, s1_ref):
    step = pl.program_id(0) * pl.num_programs(1) + pl.program_id(1)
    it = it_ref[...]
    uf = uf_ref[0]
    y = lax.dot_general(w0_ref[:, :C2], it, (((1,), (1,)), ((), ())),
                        preferred_element_type=jnp.float32)
    y = y + lax.dot_general(w0_ref[:, C2:], uf, (((1,), (0,)), ((), ())),
                            preferred_element_type=jnp.float32)
    y1_ref[0] = y
    st = jnp.concatenate([jnp.sum(y, axis=1)[None, :],
                          jnp.sum(y * y, axis=1)[None, :]], axis=0)

    @pl.when(step == 0)
    def _():
        s1_ref[...] = st

    @pl.when(step != 0)
    def _():
        s1_ref[...] += st


def _mlp1(interp, uf, w0, h):
    return pl.pallas_call(
        _mlp1_body,
        grid=(HB, NT2),
        in_specs=[
            pl.BlockSpec((TN2, C2), lambda b, t: (b * NT2 + t, 0)),
            pl.BlockSpec((1, C1, TN2), lambda b, t: (b + h * HB, 0, t)),
            pl.BlockSpec((CO, C2 + C1), lambda b, t: (0, 0)),
        ],
        out_specs=[
            pl.BlockSpec((1, CO, TN2), lambda b, t: (b, 0, t)),
            pl.BlockSpec((2, CO), lambda b, t: (0, 0)),
        ],
        out_shape=[
            jax.ShapeDtypeStruct((HB, CO, N), jnp.float32),
            jax.ShapeDtypeStruct((2, CO), jnp.float32),
        ],
    )(interp, uf, w0)


def _mlp2_body(y1_ref, sa_ref, sb_ref, g_ref, be_ref, w1_ref, y2_ref, s2_ref):
    step = pl.program_id(0) * pl.num_programs(1) + pl.program_id(1)
    n = jnp.float32(B * N)
    s0 = sa_ref[0, :] + sb_ref[0, :]
    s1 = sa_ref[1, :] + sb_ref[1, :]
    mean = s0 / n
    var = s1 / n - mean * mean
    scale = g_ref[...] / jnp.sqrt(var + 1e-5)
    shift = be_ref[...] - mean * scale
    h = jnp.maximum(y1_ref[0] * scale[:, None] + shift[:, None], 0.0)
    y = lax.dot_general(w1_ref[...], h, (((1,), (0,)), ((), ())),
                        preferred_element_type=jnp.float32)
    y2_ref[0] = y
    st = jnp.concatenate([jnp.sum(y, axis=1)[None, :],
                          jnp.sum(y * y, axis=1)[None, :]], axis=0)

    @pl.when(step == 0)
    def _():
        s2_ref[...] = st

    @pl.when(step != 0)
    def _():
        s2_ref[...] += st


def _mlp2(y1, s1a, s1b, g0, be0, w1):
    return pl.pallas_call(
        _mlp2_body,
        grid=(HB, NT2),
        in_specs=[
            pl.BlockSpec((1, CO, TN2), lambda b, t: (b, 0, t)),
            pl.BlockSpec((2, CO), lambda b, t: (0, 0)),
            pl.BlockSpec((2, CO), lambda b, t: (0, 0)),
            pl.BlockSpec((CO,), lambda b, t: (0,)),
            pl.BlockSpec((CO,), lambda b, t: (0,)),
            pl.BlockSpec((CO, CO), lambda b, t: (0, 0)),
        ],
        out_specs=[
            pl.BlockSpec((1, CO, TN2), lambda b, t: (b, 0, t)),
            pl.BlockSpec((2, CO), lambda b, t: (0, 0)),
        ],
        out_shape=[
            jax.ShapeDtypeStruct((HB, CO, N), jnp.float32),
            jax.ShapeDtypeStruct((2, CO), jnp.float32),
        ],
    )(y1, s1a, s1b, g0, be0, w1)


def _bnout_body(y2_ref, sa_ref, sb_ref, g_ref, be_ref, *rest):
    if len(rest) == 2:
        out_ref = rest[1]
    else:
        (out_ref,) = rest
    n = jnp.float32(B * N)
    s0 = sa_ref[0, :] + sb_ref[0, :]
    s1 = sa_ref[1, :] + sb_ref[1, :]
    mean = s0 / n
    var = s1 / n - mean * mean
    scale = g_ref[...] / jnp.sqrt(var + 1e-5)
    shift = be_ref[...] - mean * scale
    out_ref[0] = jnp.maximum(y2_ref[0] * scale[:, None] + shift[:, None], 0.0)


def _bnout(y2, s2a, s2b, g1, be1, prev, h):
    in_specs = [
        pl.BlockSpec((1, CO, TN2), lambda b, t: (b, 0, t)),
        pl.BlockSpec((2, CO), lambda b, t: (0, 0)),
        pl.BlockSpec((2, CO), lambda b, t: (0, 0)),
        pl.BlockSpec((CO,), lambda b, t: (0,)),
        pl.BlockSpec((CO,), lambda b, t: (0,)),
    ]
    args = [y2, s2a, s2b, g1, be1]
    aliases = {}
    if prev is not None:
        in_specs.append(pl.BlockSpec(memory_space=pl.ANY))
        args.append(prev)
        aliases = {5: 0}
    return pl.pallas_call(
        _bnout_body,
        grid=(HB, NT2),
        in_specs=in_specs,
        out_specs=pl.BlockSpec((1, CO, TN2), lambda b, t: (b + h * HB, 0, t)),
        out_shape=jax.ShapeDtypeStruct((B, CO, N), jnp.float32),
        input_output_aliases=aliases,
    )(*args)


def kernel(unknown, known, unknow_feats, known_feats, W0, g0, be0, W1, g1, be1):

    interps = []
    for h in range(2):
        i0, i1, i2, w0e, w1e, w2e, kft = _knn(unknown, known, known_feats, h)
        interps.append(_sc_interp(i0, i1, i2, w0e, w1e, w2e, kft))

    y1a, s1a = _mlp1(interps[0], unknow_feats, W0, 0)
    y1b, s1b = _mlp1(interps[1], unknow_feats, W0, 1)
    y2a, s2a = _mlp2(y1a, s1a, s1b, g0, be0, W1)
    y2b, s2b = _mlp2(y1b, s1a, s1b, g0, be0, W1)
    outa = _bnout(y2a, s2a, s2b, g1, be1, None, 0)
    return _bnout(y2b, s2a, s2b, g1, be1, outa, 1)


# speedup vs baseline: 1.0380x; 1.0380x over previous
import functools

import jax
import jax.numpy as jnp
from jax import lax
from jax.experimental import pallas as pl
from jax.experimental.pallas import tpu as pltpu
from jax.experimental.pallas import tpu_sc as plsc

B, N, M = 8, 4096, 1024
C1, C2 = 256, 512
CO = 256
HB = B // 2

TN = 1024
TN2 = 1024
NT = N // TN
NT2 = N // TN2

NW = 32
PPW = (HB * N) // NW
CH = 16
NCH = PPW // CH
K = 3


def _knn_body(u_ref, k_ref, kf_ref, gidx_ref, w_ref, kft_ref):
    b = pl.program_id(0)
    u = u_ref[0]
    kp = k_ref[0]

    @pl.when(pl.program_id(1) == 0)
    def _():
        kft_ref[...] = jnp.swapaxes(kf_ref[0], 0, 1)

    uk = lax.dot_general(u, kp, (((1,), (1,)), ((), ())),
                         preferred_element_type=jnp.float32)
    u2 = jnp.sum(u * u, axis=1, keepdims=True)
    k2 = jnp.sum(kp * kp, axis=1)[None, :]
    d = jnp.maximum(u2 + k2 - 2.0 * uk, 0.0)
    iota = lax.broadcasted_iota(jnp.int32, d.shape, 1).astype(jnp.float32)
    idxs, recips = [], []
    for k in range(K):
        mk = jnp.min(d, axis=1, keepdims=True)
        am = jnp.min(jnp.where(d == mk, iota, jnp.float32(M)),
                     axis=1, keepdims=True)
        idxs.append(am)
        recips.append(1.0 / (jnp.sqrt(mk) + 1e-8))
        if k < K - 1:
            d = jnp.where(iota == am, jnp.float32(jnp.inf), d)
    idx3 = jnp.concatenate(idxs, axis=1).astype(jnp.int32)
    norm = recips[0] + recips[1] + recips[2]
    gidx_ref[...] = idx3 + b * M
    w_ref[...] = jnp.concatenate(
        [jnp.broadcast_to(r / norm, (TN, 16)) for r in recips], axis=1)


def _knn(unknown, known, known_feats, h):
    return pl.pallas_call(
        _knn_body,
        grid=(HB, NT),
        in_specs=[
            pl.BlockSpec((1, TN, 3), lambda b, t: (b + h * HB, t, 0)),
            pl.BlockSpec((1, M, 3), lambda b, t: (b + h * HB, 0, 0)),
            pl.BlockSpec((1, C2, M), lambda b, t: (b + h * HB, 0, 0)),
        ],
        out_specs=[
            pl.BlockSpec((TN, K), lambda b, t: (b * NT + t, 0)),
            pl.BlockSpec((TN, 48), lambda b, t: (b * NT + t, 0)),
            pl.BlockSpec((M, C2), lambda b, t: (b, 0)),
        ],
        out_shape=[
            jax.ShapeDtypeStruct((HB * N, K), jnp.int32),
            jax.ShapeDtypeStruct((HB * N, 48), jnp.float32),
            jax.ShapeDtypeStruct((HB * M, C2), jnp.float32),
        ],
    )(unknown, known, known_feats)


def _sc_interp_body(gidx_hbm, w_hbm, kft_hbm, out_hbm,
                    idx_v, w_v, rows0, rows1, out0, out1,
                    gsem0, gsem1, osem0, osem1):
    wid = lax.axis_index("s") * 2 + lax.axis_index("c")
    base = wid * PPW
    pltpu.sync_copy(gidx_hbm.at[pl.ds(base * K, PPW * K)], idx_v)
    pltpu.sync_copy(w_hbm.at[pl.ds(base * 48, PPW * 48)], w_v)
    pltpu.async_copy(kft_hbm.at[idx_v.at[pl.ds(0, CH * K)]], rows0, gsem0)

    def _compute(ci, rows_v, out_v):
        p0 = ci * CH

        @plsc.parallel_loop(0, CH, 1)
        def point(j):
            gp = p0 + j
            i0 = K * j
            wo = gp * 48
            wa = w_v[pl.ds(wo, 16)]
            wb = w_v[pl.ds(wo + 16, 16)]
            wc = w_v[pl.ds(wo + 32, 16)]
            for c in range(C2 // 16):
                sl = pl.ds(c * 16, 16)
                acc = wa * rows_v[i0, sl]
                acc = acc + wb * rows_v[i0 + 1, sl]
                acc = acc + wc * rows_v[i0 + 2, sl]
                out_v[j, sl] = acc

    def _wait_gather(rows_v, sem):
        pltpu.make_async_copy(kft_hbm.at[pl.ds(0, CH * K)], rows_v, sem).wait()

    def _wait_store(out_v, sem):
        pltpu.make_async_copy(out_hbm.at[pl.ds(0, CH)], out_v, sem).wait()

    def pair(i, carry):
        c0 = 2 * i
        c1 = 2 * i + 1
        pltpu.async_copy(kft_hbm.at[idx_v.at[pl.ds(c1 * CH * K, CH * K)]],
                         rows1, gsem1)
        _wait_gather(rows0, gsem0)

        @pl.when(i > 0)
        def _():
            _wait_store(out0, osem0)

        _compute(c0, rows0, out0)
        pltpu.async_copy(out0, out_hbm.at[pl.ds(base + c0 * CH, CH)], osem0)

        @pl.when(i < NCH // 2 - 1)
        def _():
            pltpu.async_copy(
                kft_hbm.at[idx_v.at[pl.ds((c0 + 2) * CH * K, CH * K)]],
                rows0, gsem0)

        _wait_gather(rows1, gsem1)

        @pl.when(i > 0)
        def _():
            _wait_store(out1, osem1)

        _compute(c1, rows1, out1)
        pltpu.async_copy(out1, out_hbm.at[pl.ds(base + c1 * CH, CH)], osem1)
        return carry

    lax.fori_loop(0, NCH // 2, pair, 0)
    _wait_store(out0, osem0)
    _wait_store(out1, osem1)


def _sc_interp(gidx_flat, w48, kft):
    mesh = plsc.VectorSubcoreMesh(core_axis_name="c", subcore_axis_name="s")
    f = functools.partial(
        pl.kernel,
        out_type=jax.ShapeDtypeStruct((HB * N, C2), jnp.float32),
        mesh=mesh,
        scratch_types=[
            pltpu.VMEM((PPW * K,), jnp.int32),
            pltpu.VMEM((PPW * 48,), jnp.float32),
            pltpu.VMEM((CH * K, C2), jnp.float32),
            pltpu.VMEM((CH * K, C2), jnp.float32),
            pltpu.VMEM((CH, C2), jnp.float32),
            pltpu.VMEM((CH, C2), jnp.float32),
            pltpu.SemaphoreType.DMA,
            pltpu.SemaphoreType.DMA,
            pltpu.SemaphoreType.DMA,
            pltpu.SemaphoreType.DMA,
        ],
    )(_sc_interp_body)
    return f(gidx_flat, w48, kft)


def _mlp1_body(it_ref, uf_ref, w0_ref, y1_ref, s1_ref):
    step = pl.program_id(0) * pl.num_programs(1) + pl.program_id(1)
    it = it_ref[...]
    uf = uf_ref[0]
    y = lax.dot_general(w0_ref[:, :C2], it, (((1,), (1,)), ((), ())),
                        preferred_element_type=jnp.float32)
    y = y + lax.dot_general(w0_ref[:, C2:], uf, (((1,), (0,)), ((), ())),
                            preferred_element_type=jnp.float32)
    y1_ref[0] = y
    st = jnp.concatenate([jnp.sum(y, axis=1)[None, :],
                          jnp.sum(y * y, axis=1)[None, :]], axis=0)

    @pl.when(step == 0)
    def _():
        s1_ref[...] = st

    @pl.when(step != 0)
    def _():
        s1_ref[...] += st


def _mlp1(interp, uf, w0, h):
    return pl.pallas_call(
        _mlp1_body,
        grid=(HB, NT2),
        in_specs=[
            pl.BlockSpec((TN2, C2), lambda b, t: (b * NT2 + t, 0)),
            pl.BlockSpec((1, C1, TN2), lambda b, t: (b + h * HB, 0, t)),
            pl.BlockSpec((CO, C2 + C1), lambda b, t: (0, 0)),
        ],
        out_specs=[
            pl.BlockSpec((1, CO, TN2), lambda b, t: (b, 0, t)),
            pl.BlockSpec((2, CO), lambda b, t: (0, 0)),
        ],
        out_shape=[
            jax.ShapeDtypeStruct((HB, CO, N), jnp.float32),
            jax.ShapeDtypeStruct((2, CO), jnp.float32),
        ],
    )(interp, uf, w0)


def _mlp2_body(y1_ref, sa_ref, sb_ref, g_ref, be_ref, w1_ref, y2_ref, s2_ref):
    step = pl.program_id(0) * pl.num_programs(1) + pl.program_id(1)
    n = jnp.float32(B * N)
    s0 = sa_ref[0, :] + sb_ref[0, :]
    s1 = sa_ref[1, :] + sb_ref[1, :]
    mean = s0 / n
    var = s1 / n - mean * mean
    scale = g_ref[...] / jnp.sqrt(var + 1e-5)
    shift = be_ref[...] - mean * scale
    h = jnp.maximum(y1_ref[0] * scale[:, None] + shift[:, None], 0.0)
    y = lax.dot_general(w1_ref[...], h, (((1,), (0,)), ((), ())),
                        preferred_element_type=jnp.float32)
    y2_ref[0] = y
    st = jnp.concatenate([jnp.sum(y, axis=1)[None, :],
                          jnp.sum(y * y, axis=1)[None, :]], axis=0)

    @pl.when(step == 0)
    def _():
        s2_ref[...] = st

    @pl.when(step != 0)
    def _():
        s2_ref[...] += st


def _mlp2(y1, s1a, s1b, g0, be0, w1):
    return pl.pallas_call(
        _mlp2_body,
        grid=(HB, NT2),
        in_specs=[
            pl.BlockSpec((1, CO, TN2), lambda b, t: (b, 0, t)),
            pl.BlockSpec((2, CO), lambda b, t: (0, 0)),
            pl.BlockSpec((2, CO), lambda b, t: (0, 0)),
            pl.BlockSpec((CO,), lambda b, t: (0,)),
            pl.BlockSpec((CO,), lambda b, t: (0,)),
            pl.BlockSpec((CO, CO), lambda b, t: (0, 0)),
        ],
        out_specs=[
            pl.BlockSpec((1, CO, TN2), lambda b, t: (b, 0, t)),
            pl.BlockSpec((2, CO), lambda b, t: (0, 0)),
        ],
        out_shape=[
            jax.ShapeDtypeStruct((HB, CO, N), jnp.float32),
            jax.ShapeDtypeStruct((2, CO), jnp.float32),
        ],
    )(y1, s1a, s1b, g0, be0, w1)


def _bnout_body(y2_ref, sa_ref, sb_ref, g_ref, be_ref, *rest):
    if len(rest) == 2:
        out_ref = rest[1]
    else:
        (out_ref,) = rest
    n = jnp.float32(B * N)
    s0 = sa_ref[0, :] + sb_ref[0, :]
    s1 = sa_ref[1, :] + sb_ref[1, :]
    mean = s0 / n
    var = s1 / n - mean * mean
    scale = g_ref[...] / jnp.sqrt(var + 1e-5)
    shift = be_ref[...] - mean * scale
    out_ref[0] = jnp.maximum(y2_ref[0] * scale[:, None] + shift[:, None], 0.0)


def _bnout(y2, s2a, s2b, g1, be1, prev, h):
    in_specs = [
        pl.BlockSpec((1, CO, TN2), lambda b, t: (b, 0, t)),
        pl.BlockSpec((2, CO), lambda b, t: (0, 0)),
        pl.BlockSpec((2, CO), lambda b, t: (0, 0)),
        pl.BlockSpec((CO,), lambda b, t: (0,)),
        pl.BlockSpec((CO,), lambda b, t: (0,)),
    ]
    args = [y2, s2a, s2b, g1, be1]
    aliases = {}
    if prev is not None:
        in_specs.append(pl.BlockSpec(memory_space=pl.ANY))
        args.append(prev)
        aliases = {5: 0}
    return pl.pallas_call(
        _bnout_body,
        grid=(HB, NT2),
        in_specs=in_specs,
        out_specs=pl.BlockSpec((1, CO, TN2), lambda b, t: (b + h * HB, 0, t)),
        out_shape=jax.ShapeDtypeStruct((B, CO, N), jnp.float32),
        input_output_aliases=aliases,
    )(*args)


def kernel(unknown, known, unknow_feats, known_feats, W0, g0, be0, W1, g1, be1):

    interps = []
    for h in range(2):
        gidx, w48, kft = _knn(unknown, known, known_feats, h)
        interps.append(_sc_interp(gidx.reshape(HB * N * K),
                                  w48.reshape(HB * N * 48), kft))

    y1a, s1a = _mlp1(interps[0], unknow_feats, W0, 0)
    y1b, s1b = _mlp1(interps[1], unknow_feats, W0, 1)
    y2a, s2a = _mlp2(y1a, s1a, s1b, g0, be0, W1)
    y2b, s2b = _mlp2(y1b, s1a, s1b, g0, be0, W1)
    outa = _bnout(y2a, s2a, s2b, g1, be1, None, 0)
    return _bnout(y2b, s2a, s2b, g1, be1, outa, 1)
